# baseline (speedup 1.0000x reference)
import functools

import jax
import jax.numpy as jnp
from jax import lax
from jax.experimental import pallas as pl
from jax.experimental.pallas import tpu as pltpu
from jax.experimental.pallas import tpu_sc as plsc

DIM = 128
K = 65536
BATCH = 4096
NW = 32
CW = 384
NCH = (K - BATCH) // (CW * NW)
WIN_W = BATCH // NW
NBUF = 2
DEPTH = 1


def _tr_body(keys_ref, out_ref):
    out_ref[...] = keys_ref[...].T


def _transpose_tc(keys):
    return pl.pallas_call(
        _tr_body,
        grid=(BATCH // 512,),
        in_specs=[pl.BlockSpec((512, DIM), lambda i: (i, 0))],
        out_specs=pl.BlockSpec((DIM, 512), lambda i: (0, i)),
        out_shape=jax.ShapeDtypeStruct((DIM, BATCH), jnp.float32),
    )(keys)


def _sc_body(keyst_hbm, q_hbm, ptr_hbm, out_hbm, ptr_out_hbm,
             bufs, tv, pv, isems, osems, wsem, wsem2):
    wid = lax.axis_index("s") * 2 + lax.axis_index("c")

    pltpu.sync_copy(ptr_hbm, pv)
    pvec = pv[...]
    p_raw = pvec[0]
    p = pl.multiple_of(jnp.clip(p_raw, 0, K - BATCH), CW)

    def col_of(i):
        base = (wid * NCH + i) * CW
        return pl.multiple_of(jnp.where(base >= p, base + BATCH, base), CW)

    def start_in(i):
        c = pltpu.make_async_copy(
            q_hbm.at[:, pl.ds(col_of(i), CW)], bufs.at[i % NBUF], isems.at[i % NBUF]
        )
        c.start()
        return c

    def start_out(i):
        c = pltpu.make_async_copy(
            bufs.at[i % NBUF], out_hbm.at[:, pl.ds(col_of(i), CW)], osems.at[i % NBUF]
        )
        c.start()
        return c

    ins = {}
    outs = {}
    out_waited = set()
    for i in range(min(DEPTH, NCH)):
        ins[i] = start_in(i)

    kin = pltpu.make_async_copy(
        keyst_hbm.at[:, pl.ds(wid * WIN_W, WIN_W)], tv, wsem
    )
    kin.start()

    for i in range(NCH):
        ins[i].wait()
        outs[i] = start_out(i)
        j = i + DEPTH
        if j < NCH:
            if j - NBUF >= 0:
                outs[j - NBUF].wait()
                out_waited.add(j - NBUF)
            ins[j] = start_in(j)

    kin.wait()
    wout = pltpu.make_async_copy(
        tv, out_hbm.at[:, pl.ds(p + wid * WIN_W, WIN_W)], wsem2
    )
    wout.start()

    @pl.when(wid == 0)
    def _():
        new_ptr = lax.rem(pvec + BATCH, jnp.full((16,), K, jnp.int32))
        pv[...] = new_ptr
        pltpu.sync_copy(pv.at[pl.ds(0, 1)], ptr_out_hbm)

    for i in range(NCH):
        if i not in out_waited:
            outs[i].wait()
    wout.wait()


def kernel(keys, queue, queue_ptr):
    keyst = _transpose_tc(keys)
    ptr_vec = jnp.broadcast_to(queue_ptr.astype(jnp.int32), (16,))
    mesh = plsc.VectorSubcoreMesh(core_axis_name="c", subcore_axis_name="s")
    sc = functools.partial(
        pl.kernel,
        out_type=[
            jax.ShapeDtypeStruct((DIM, K), jnp.float32),
            jax.ShapeDtypeStruct((1,), jnp.int32),
        ],
        mesh=mesh,
        scratch_types=[
            pltpu.VMEM((NBUF, DIM, CW), jnp.float32),
            pltpu.VMEM((DIM, WIN_W), jnp.float32),
            pltpu.VMEM((16,), jnp.int32),
            pltpu.SemaphoreType.DMA((NBUF,)),
            pltpu.SemaphoreType.DMA((NBUF,)),
            pltpu.SemaphoreType.DMA,
            pltpu.SemaphoreType.DMA,
        ],
    )(_sc_body)
    new_queue, new_ptr = sc(keyst, queue, ptr_vec)
    return new_queue, new_ptr.astype(queue_ptr.dtype)


# speedup vs baseline: 2.1102x; 2.1102x over previous
import jax
import jax.numpy as jnp
from jax.experimental import pallas as pl
from jax.experimental.pallas import tpu as pltpu

DIM = 128
K = 65536
BATCH = 4096
CHUNK = 2048
NCH = (K - BATCH) // CHUNK
NBUF = 30
DEPTH = 30


def _body(ptr_ref, keys_ref, q_ref, out_ref, ptr_out_ref, bufs, tv, isem, osem, wsem):
    p = jnp.clip(ptr_ref[0], 0, K - BATCH)
    p = pl.multiple_of(p, BATCH)

    def col_of(i):
        base = i * CHUNK
        return pl.multiple_of(jnp.where(base >= p, base + BATCH, base), CHUNK)

    def start_in(i):
        b = i % NBUF
        c = pltpu.make_async_copy(
            q_ref.at[:, pl.ds(col_of(i), CHUNK)], bufs.at[b], isem.at[b]
        )
        c.start()
        return c

    def start_out(i):
        b = i % NBUF
        c = pltpu.make_async_copy(
            bufs.at[b], out_ref.at[:, pl.ds(col_of(i), CHUNK)], osem.at[b]
        )
        c.start()
        return c

    ins = {}
    outs = {}
    for i in range(DEPTH):
        ins[i] = start_in(i)

    def tr(i, carry):
        tv[:, pl.ds(i * DIM, DIM)] = keys_ref[pl.ds(i * DIM, DIM), :].T
        return carry

    jax.lax.fori_loop(0, BATCH // DIM, tr, 0)
    w = pltpu.make_async_copy(tv, out_ref.at[:, pl.ds(p, BATCH)], wsem)
    w.start()

    ptr_out_ref[0] = jax.lax.rem(ptr_ref[0] + BATCH, K)

    for i in range(NCH):
        ins[i].wait()
        outs[i] = start_out(i)
        j = i + DEPTH
        if j < NCH:
            if j - NBUF >= 0:
                outs[j - NBUF].wait()
            ins[j] = start_in(j)

    for i in range(max(0, NCH - NBUF), NCH):
        outs[i].wait()
    w.wait()


def kernel(keys, queue, queue_ptr):
    ptr = queue_ptr.astype(jnp.int32)
    new_queue, new_ptr = pl.pallas_call(
        _body,
        grid=(),
        in_specs=[
            pl.BlockSpec(memory_space=pltpu.SMEM),
            pl.BlockSpec(memory_space=pltpu.VMEM),
            pl.BlockSpec(memory_space=pl.ANY),
        ],
        out_specs=[
            pl.BlockSpec(memory_space=pl.ANY),
            pl.BlockSpec(memory_space=pltpu.SMEM),
        ],
        out_shape=[
            jax.ShapeDtypeStruct((DIM, K), jnp.float32),
            jax.ShapeDtypeStruct((1,), jnp.int32),
        ],
        scratch_shapes=[
            pltpu.VMEM((NBUF, DIM, CHUNK), jnp.float32),
            pltpu.VMEM((DIM, BATCH), jnp.float32),
            pltpu.SemaphoreType.DMA((NBUF,)),
            pltpu.SemaphoreType.DMA((NBUF,)),
            pltpu.SemaphoreType.DMA,
        ],
    )(ptr, keys, queue)
    return new_queue, new_ptr.astype(queue_ptr.dtype)


# speedup vs baseline: 2.1398x; 1.0140x over previous
import jax
import jax.numpy as jnp
from jax.experimental import pallas as pl
from jax.experimental.pallas import tpu as pltpu

DIM = 128
K = 65536
BATCH = 4096
CHUNK = 4096
NCH = (K - BATCH) // CHUNK
NBUF = 15
DEPTH = 15


def _body(ptr_ref, keys_ref, q_ref, out_ref, ptr_out_ref, bufs, tv, isem, osem, wsem):
    p = jnp.clip(ptr_ref[0], 0, K - BATCH)
    p = pl.multiple_of(p, BATCH)

    def col_of(i):
        base = i * CHUNK
        return pl.multiple_of(jnp.where(base >= p, base + BATCH, base), CHUNK)

    def start_in(i):
        b = i % NBUF
        c = pltpu.make_async_copy(
            q_ref.at[:, pl.ds(col_of(i), CHUNK)], bufs.at[b], isem.at[b]
        )
        c.start()
        return c

    def start_out(i):
        b = i % NBUF
        c = pltpu.make_async_copy(
            bufs.at[b], out_ref.at[:, pl.ds(col_of(i), CHUNK)], osem.at[b]
        )
        c.start()
        return c

    ins = {}
    outs = {}
    for i in range(DEPTH):
        ins[i] = start_in(i)

    def tr(i, carry):
        tv[:, pl.ds(i * DIM, DIM)] = keys_ref[pl.ds(i * DIM, DIM), :].T
        return carry

    jax.lax.fori_loop(0, BATCH // DIM, tr, 0)
    w = pltpu.make_async_copy(tv, out_ref.at[:, pl.ds(p, BATCH)], wsem)
    w.start()

    ptr_out_ref[0] = jax.lax.rem(ptr_ref[0] + BATCH, K)

    for i in range(NCH):
        ins[i].wait()
        outs[i] = start_out(i)
        j = i + DEPTH
        if j < NCH:
            if j - NBUF >= 0:
                outs[j - NBUF].wait()
            ins[j] = start_in(j)

    for i in range(max(0, NCH - NBUF), NCH):
        outs[i].wait()
    w.wait()


def kernel(keys, queue, queue_ptr):
    ptr = queue_ptr.astype(jnp.int32)
    new_queue, new_ptr = pl.pallas_call(
        _body,
        grid=(),
        in_specs=[
            pl.BlockSpec(memory_space=pltpu.SMEM),
            pl.BlockSpec(memory_space=pltpu.VMEM),
            pl.BlockSpec(memory_space=pl.ANY),
        ],
        out_specs=[
            pl.BlockSpec(memory_space=pl.ANY),
            pl.BlockSpec(memory_space=pltpu.SMEM),
        ],
        out_shape=[
            jax.ShapeDtypeStruct((DIM, K), jnp.float32),
            jax.ShapeDtypeStruct((1,), jnp.int32),
        ],
        scratch_shapes=[
            pltpu.VMEM((NBUF, DIM, CHUNK), jnp.float32),
            pltpu.VMEM((DIM, BATCH), jnp.float32),
            pltpu.SemaphoreType.DMA((NBUF,)),
            pltpu.SemaphoreType.DMA((NBUF,)),
            pltpu.SemaphoreType.DMA,
        ],
    )(ptr, keys, queue)
    return new_queue, new_ptr.astype(queue_ptr.dtype)
